# baseline (speedup 1.0000x reference)
import functools

import jax
import jax.numpy as jnp
from jax import lax
from jax.experimental import pallas as pl
from jax.experimental.pallas import tpu as pltpu
from jax.experimental.pallas import tpu_sc as plsc

_VOCAB = 1000
_EMB = 32
_T = 8
_B = 4096
_NROWS = _B * _T
_VPAD = 1024
_NSUB = _VPAD // 128

_NW = 32
_ROWS_PER_W = _NROWS // _NW
_CHUNK = 16
_NCHUNK = _ROWS_PER_W // _CHUNK
_BPC = _CHUNK // _T

_NVREG = _VOCAB // 16
_TAILO = _NVREG * 16


def _table_kernel(emb_ref, w_ref, out_ref):
    out_ref[:] = jnp.dot(emb_ref[:], w_ref[:],
                         preferred_element_type=jnp.float32)


def _build_table(embedding, lm_head_w):
    w_pad = jnp.pad(lm_head_w, ((0, 0), (0, _VPAD - _VOCAB)))
    return pl.pallas_call(
        _table_kernel,
        grid=(_NSUB,),
        in_specs=[
            pl.BlockSpec((_VOCAB, _EMB), lambda c: (0, 0)),
            pl.BlockSpec((_EMB, 128), lambda c: (0, c)),
        ],
        out_specs=pl.BlockSpec((_VOCAB, 128), lambda c: (c, 0)),
        out_shape=jax.ShapeDtypeStruct((_NSUB * _VOCAB, 128), jnp.float32),
    )(embedding, w_pad)


def _repack_chunk(bufg, bufo, posv):
    lanes = lax.iota(jnp.int32, 16)
    tail_lane = _TAILO + lanes
    tail_mask = lanes < (_VOCAB - _TAILO)
    zeros = jnp.zeros((16,), jnp.int32)

    def row_body(r, _):
        b_loc = r // _T
        t = r % _T
        for j in range(_NVREG):
            o = 16 * j
            val = bufg[_NSUB * r + o // 128, pl.ds(o % 128, 16)]
            val = val + posv[t, pl.ds(o, 16)]
            bufo[b_loc, t, pl.ds(o, 16)] = val
        val = bufg[_NSUB * r + _TAILO // 128, pl.ds(_TAILO % 128, 16)]
        val = val + posv[t, pl.ds(_TAILO, 16)]
        plsc.store_scatter(bufo, [zeros + b_loc, zeros + t, tail_lane],
                           val, mask=tail_mask)
        return ()

    lax.fori_loop(0, _CHUNK, row_body, ())


def _sc_gather_body(tab_ref, posw_ref, jidx_ref, out_ref,
                    idx_v, posv, bufg0, bufg1, bufo0, bufo1,
                    gsem0, gsem1, osem0, osem1):
    wid = lax.axis_index("s") * 2 + lax.axis_index("c")
    base_b = wid * (_ROWS_PER_W // _T)

    pltpu.sync_copy(jidx_ref.at[wid], idx_v)
    pltpu.sync_copy(posw_ref, posv)

    bufgs = (bufg0, bufg1)
    bufos = (bufo0, bufo1)
    gsems = (gsem0, gsem1)
    osems = (osem0, osem1)

    def gather(c, p):
        pltpu.async_copy(tab_ref.at[idx_v.at[c]], bufgs[p], gsems[p])

    def wait_gather(p):
        pltpu.make_async_copy(tab_ref.at[pl.ds(0, _CHUNK * _NSUB)],
                              bufgs[p], gsems[p]).wait()

    def out_copy(c, p):
        pltpu.async_copy(
            bufos[p],
            out_ref.at[pl.ds(base_b + c * _BPC, _BPC)],
            osems[p],
        )

    def wait_out(p):
        pltpu.make_async_copy(out_ref.at[pl.ds(0, _BPC)],
                              bufos[p], osems[p]).wait()

    gather(0, 0)
    gather(1, 1)

    def pair_body(k, _):
        c0 = 2 * k
        for p in range(2):
            c = c0 + p
            wait_gather(p)
            pl.when(k > 0)(lambda: wait_out(p))
            _repack_chunk(bufgs[p], bufos[p], posv)
            pl.when(c + 2 < _NCHUNK)(lambda: gather(c + 2, p))
            out_copy(c, p)
        return ()

    lax.fori_loop(0, _NCHUNK // 2, pair_body, ())
    wait_out(0)
    wait_out(1)


def _sc_gather(table, posw, jidx):
    mesh = plsc.VectorSubcoreMesh(core_axis_name="c", subcore_axis_name="s")
    fn = functools.partial(
        pl.kernel,
        out_type=jax.ShapeDtypeStruct((_B, _T, _VOCAB), jnp.float32),
        mesh=mesh,
        scratch_types=[
            pltpu.VMEM((_NCHUNK, _CHUNK * _NSUB), jnp.int32),
            pltpu.VMEM((_T, _VPAD), jnp.float32),
            pltpu.VMEM((_CHUNK * _NSUB, 128), jnp.float32),
            pltpu.VMEM((_CHUNK * _NSUB, 128), jnp.float32),
            pltpu.VMEM((_BPC, _T, _VOCAB), jnp.float32),
            pltpu.VMEM((_BPC, _T, _VOCAB), jnp.float32),
            pltpu.SemaphoreType.DMA,
            pltpu.SemaphoreType.DMA,
            pltpu.SemaphoreType.DMA,
            pltpu.SemaphoreType.DMA,
        ],
        compiler_params=pltpu.CompilerParams(needs_layout_passes=False),
    )(_sc_gather_body)
    return fn(table, posw, jidx)


@jax.jit
def kernel(idx, embedding, positional_embedding, lm_head_w, lm_head_b):
    table = _build_table(embedding, lm_head_w)
    posw = positional_embedding @ lm_head_w + lm_head_b[None, :]
    posw = jnp.pad(posw, ((0, 0), (0, _VPAD - _VOCAB)))
    flat = idx.reshape(_NROWS).astype(jnp.int32)
    s = jax.lax.iota(jnp.int32, _NW * _NCHUNK * _CHUNK * _NSUB)
    i = s // _NSUB
    c = s % _NSUB
    j8 = c * _VOCAB + flat[i]
    jidx = j8.reshape(_NW, _NCHUNK, _CHUNK * _NSUB)
    return _sc_gather(table, posw, jidx)


# speedup vs baseline: 1.0605x; 1.0605x over previous
import functools

import jax
import jax.numpy as jnp
from jax import lax
from jax.experimental import pallas as pl
from jax.experimental.pallas import tpu as pltpu
from jax.experimental.pallas import tpu_sc as plsc

_VOCAB = 1000
_EMB = 32
_T = 8
_B = 4096
_NROWS = _B * _T
_VPAD = 1024
_NSUB = _VPAD // 128

_NW = 32
_ROWS_PER_W = _NROWS // _NW
_CHUNK = 8
_NCHUNK = _ROWS_PER_W // _CHUNK
_BPC = _CHUNK // _T

_NVREG = _VOCAB // 16
_TAILO = _NVREG * 16


def _table_kernel(emb_ref, w_ref, out_ref):
    out_ref[:] = jnp.dot(emb_ref[:], w_ref[:],
                         preferred_element_type=jnp.float32)


def _build_table(embedding, lm_head_w):
    w_pad = jnp.pad(lm_head_w, ((0, 0), (0, _VPAD - _VOCAB)))
    return pl.pallas_call(
        _table_kernel,
        grid=(_NSUB,),
        in_specs=[
            pl.BlockSpec((_VOCAB, _EMB), lambda c: (0, 0)),
            pl.BlockSpec((_EMB, 128), lambda c: (0, c)),
        ],
        out_specs=pl.BlockSpec((_VOCAB, 128), lambda c: (c, 0)),
        out_shape=jax.ShapeDtypeStruct((_NSUB * _VOCAB, 128), jnp.float32),
    )(embedding, w_pad)


def _repack_chunk(bufg, bufo, posv):
    lanes = lax.iota(jnp.int32, 16)
    tail_lane = _TAILO + lanes
    tail_mask = lanes < (_VOCAB - _TAILO)
    zeros = jnp.zeros((16,), jnp.int32)

    for r in range(_CHUNK):
        t = r % _T
        for j in range(_NVREG):
            o = 16 * j
            val = bufg[_NSUB * r + o // 128, pl.ds(o % 128, 16)]
            val = val + posv[t, pl.ds(o, 16)]
            bufo[0, t, pl.ds(o, 16)] = val
        val = bufg[_NSUB * r + _TAILO // 128, pl.ds(_TAILO % 128, 16)]
        val = val + posv[t, pl.ds(_TAILO, 16)]
        plsc.store_scatter(bufo, [zeros, zeros + t, tail_lane],
                           val, mask=tail_mask)


def _sc_gather_body(tab_ref, posw_ref, jidx_ref, out_ref,
                    idx_v, posv, bufg0, bufg1, bufo0, bufo1,
                    gsem0, gsem1, osem0, osem1):
    wid = lax.axis_index("s") * 2 + lax.axis_index("c")
    base_b = wid * (_ROWS_PER_W // _T)

    pltpu.sync_copy(jidx_ref.at[wid], idx_v)
    pltpu.sync_copy(posw_ref, posv)

    bufgs = (bufg0, bufg1)
    bufos = (bufo0, bufo1)
    gsems = (gsem0, gsem1)
    osems = (osem0, osem1)

    def gather(c, p):
        pltpu.async_copy(tab_ref.at[idx_v.at[c]], bufgs[p], gsems[p])

    def wait_gather(p):
        pltpu.make_async_copy(tab_ref.at[pl.ds(0, _CHUNK * _NSUB)],
                              bufgs[p], gsems[p]).wait()

    def out_copy(c, p):
        pltpu.async_copy(
            bufos[p],
            out_ref.at[pl.ds(base_b + c * _BPC, _BPC)],
            osems[p],
        )

    def wait_out(p):
        pltpu.make_async_copy(out_ref.at[pl.ds(0, _BPC)],
                              bufos[p], osems[p]).wait()

    gather(0, 0)
    gather(1, 1)

    def pair_body(k, _):
        c0 = 2 * k
        for p in range(2):
            c = c0 + p
            wait_gather(p)
            pl.when(k > 0)(lambda: wait_out(p))
            _repack_chunk(bufgs[p], bufos[p], posv)
            pl.when(c + 2 < _NCHUNK)(lambda: gather(c + 2, p))
            out_copy(c, p)
        return ()

    lax.fori_loop(0, _NCHUNK // 2, pair_body, ())
    wait_out(0)
    wait_out(1)


def _sc_gather(table, posw, jidx):
    mesh = plsc.VectorSubcoreMesh(core_axis_name="c", subcore_axis_name="s")
    fn = functools.partial(
        pl.kernel,
        out_type=jax.ShapeDtypeStruct((_B, _T, _VOCAB), jnp.float32),
        mesh=mesh,
        scratch_types=[
            pltpu.VMEM((_NCHUNK, _CHUNK * _NSUB), jnp.int32),
            pltpu.VMEM((_T, _VPAD), jnp.float32),
            pltpu.VMEM((_CHUNK * _NSUB, 128), jnp.float32),
            pltpu.VMEM((_CHUNK * _NSUB, 128), jnp.float32),
            pltpu.VMEM((_BPC, _T, _VOCAB), jnp.float32),
            pltpu.VMEM((_BPC, _T, _VOCAB), jnp.float32),
            pltpu.SemaphoreType.DMA,
            pltpu.SemaphoreType.DMA,
            pltpu.SemaphoreType.DMA,
            pltpu.SemaphoreType.DMA,
        ],
        compiler_params=pltpu.CompilerParams(needs_layout_passes=False),
    )(_sc_gather_body)
    return fn(table, posw, jidx)


@jax.jit
def kernel(idx, embedding, positional_embedding, lm_head_w, lm_head_b):
    table = _build_table(embedding, lm_head_w)
    posw = positional_embedding @ lm_head_w + lm_head_b[None, :]
    posw = jnp.pad(posw, ((0, 0), (0, _VPAD - _VOCAB)))
    flat = idx.reshape(_NROWS).astype(jnp.int32)
    s = jax.lax.iota(jnp.int32, _NW * _NCHUNK * _CHUNK * _NSUB)
    i = s // _NSUB
    c = s % _NSUB
    j8 = c * _VOCAB + flat[i]
    jidx = j8.reshape(_NW, _NCHUNK, _CHUNK * _NSUB)
    return _sc_gather(table, posw, jidx)


# speedup vs baseline: 6.7096x; 6.3267x over previous
import functools

import jax
import jax.numpy as jnp
from jax import lax
from jax.experimental import pallas as pl
from jax.experimental.pallas import tpu as pltpu
from jax.experimental.pallas import tpu_sc as plsc

_VOCAB = 1000
_EMB = 32
_T = 8
_B = 4096
_NROWS = _B * _T
_VPAD = 1024
_NSUB = _VPAD // 128

_NW = 32
_ROWS_PER_W = _NROWS // _NW
_CHUNK = 8
_NCHUNK = _ROWS_PER_W // _CHUNK
_BPC = _CHUNK // _T

_NVREG = _VOCAB // 16
_TAILO = _NVREG * 16


def _table_kernel(emb_ref, w_ref, out_ref):
    out_ref[:] = jnp.dot(emb_ref[:], w_ref[:],
                         preferred_element_type=jnp.float32)


def _build_table(embedding, lm_head_w):
    w_pad = jnp.pad(lm_head_w, ((0, 0), (0, _VPAD - _VOCAB)))
    return pl.pallas_call(
        _table_kernel,
        grid=(1,),
        in_specs=[
            pl.BlockSpec((_VOCAB, _EMB), lambda c: (0, 0)),
            pl.BlockSpec((_EMB, _VPAD), lambda c: (0, 0)),
        ],
        out_specs=pl.BlockSpec((_VOCAB, _VPAD), lambda c: (0, 0)),
        out_shape=jax.ShapeDtypeStruct((_VOCAB, _VPAD), jnp.float32),
    )(embedding, w_pad)


def _repack_chunk(bufg, bufo, posv):
    lanes = lax.iota(jnp.int32, 16)
    tail_lane = _TAILO + lanes
    tail_mask = lanes < (_VOCAB - _TAILO)
    zeros = jnp.zeros((16,), jnp.int32)

    for r in range(_CHUNK):
        t = r % _T
        for j in range(_NVREG):
            o = 16 * j
            val = bufg[r, pl.ds(o, 16)]
            val = val + posv[t, pl.ds(o, 16)]
            bufo[0, t, pl.ds(o, 16)] = val
        val = bufg[r, pl.ds(_TAILO, 16)]
        val = val + posv[t, pl.ds(_TAILO, 16)]
        plsc.store_scatter(bufo, [zeros, zeros + t, tail_lane],
                           val, mask=tail_mask)


def _sc_gather_body(tab_ref, posw_ref, jidx_ref, out_ref,
                    idx_v, posv, bufg0, bufg1, bufo0, bufo1,
                    gsem0, gsem1, osem0, osem1):
    wid = lax.axis_index("s") * 2 + lax.axis_index("c")
    base_b = wid * (_ROWS_PER_W // _T)

    pltpu.sync_copy(jidx_ref.at[wid], idx_v)
    pltpu.sync_copy(posw_ref, posv)

    bufgs = (bufg0, bufg1)
    bufos = (bufo0, bufo1)
    gsems = (gsem0, gsem1)
    osems = (osem0, osem1)

    def gather(c, p):
        pltpu.async_copy(tab_ref.at[idx_v.at[c]], bufgs[p], gsems[p])

    def wait_gather(p):
        pltpu.make_async_copy(tab_ref.at[pl.ds(0, _CHUNK)],
                              bufgs[p], gsems[p]).wait()

    def out_copy(c, p):
        pltpu.async_copy(
            bufos[p],
            out_ref.at[pl.ds(base_b + c * _BPC, _BPC)],
            osems[p],
        )

    def wait_out(p):
        pltpu.make_async_copy(out_ref.at[pl.ds(0, _BPC)],
                              bufos[p], osems[p]).wait()

    gather(0, 0)
    gather(1, 1)

    def pair_body(k, _):
        c0 = 2 * k
        for p in range(2):
            c = c0 + p
            wait_gather(p)
            pl.when(k > 0)(lambda: wait_out(p))
            _repack_chunk(bufgs[p], bufos[p], posv)
            pl.when(c + 2 < _NCHUNK)(lambda: gather(c + 2, p))
            out_copy(c, p)
        return ()

    lax.fori_loop(0, _NCHUNK // 2, pair_body, ())
    wait_out(0)
    wait_out(1)


def _sc_gather(table, posw, jidx):
    mesh = plsc.VectorSubcoreMesh(core_axis_name="c", subcore_axis_name="s")
    fn = functools.partial(
        pl.kernel,
        out_type=jax.ShapeDtypeStruct((_B, _T, _VOCAB), jnp.float32),
        mesh=mesh,
        scratch_types=[
            pltpu.VMEM((_NCHUNK, _CHUNK), jnp.int32),
            pltpu.VMEM((_T, _VPAD), jnp.float32),
            pltpu.VMEM((_CHUNK, _VPAD), jnp.float32),
            pltpu.VMEM((_CHUNK, _VPAD), jnp.float32),
            pltpu.VMEM((_BPC, _T, _VOCAB), jnp.float32),
            pltpu.VMEM((_BPC, _T, _VOCAB), jnp.float32),
            pltpu.SemaphoreType.DMA,
            pltpu.SemaphoreType.DMA,
            pltpu.SemaphoreType.DMA,
            pltpu.SemaphoreType.DMA,
        ],
        compiler_params=pltpu.CompilerParams(needs_layout_passes=False),
    )(_sc_gather_body)
    return fn(table, posw, jidx)


@jax.jit
def kernel(idx, embedding, positional_embedding, lm_head_w, lm_head_b):
    table = _build_table(embedding, lm_head_w)
    posw = positional_embedding @ lm_head_w + lm_head_b[None, :]
    posw = jnp.pad(posw, ((0, 0), (0, _VPAD - _VOCAB)))
    flat = idx.reshape(_NROWS).astype(jnp.int32)
    jidx = flat.reshape(_NW, _NCHUNK, _CHUNK)
    return _sc_gather(table, posw, jidx)
